# baseline (device time: 34730 ns/iter reference)
import jax
import jax.numpy as jnp
from jax import lax
from jax.experimental import pallas as pl
from jax.experimental.pallas import tpu as pltpu

N_DEV = 16

RING = (0, 1, 5, 9, 13, 14, 10, 6, 2, 3, 7, 11, 15, 12, 8, 4)

S = 2
MAX_HOP = 8

def QROW(r):
    return (r - 6) % N_DEV

_PIECES = ((0, 128), (128, 256), (384, 512), (256, 384))

_STREAMS = (
    (8, 0, "R", lambda h: 8 - h, lambda h: 7 - h),
    (7, 0, "L", lambda h: 9 + h, lambda h: 10 + h),
    (8, 1, "L", lambda h: 8 + h, lambda h: 9 + h),
    (7, 1, "R", lambda h: 7 - h, lambda h: 6 - h),
)


def kernel(x, dy):
    m, d = x.shape
    _, f = dy.shape
    chunk = d // N_DEV
    hrow = chunk // 2
    rh = hrow // S

    def body(x_ref, dy_ref, out_ref, xt_ref, xtp_ref, acc_ref, comm_ref,
             send_sems, recv_sems):
        my = lax.axis_index("i")

        ind = [(my == RING[k]).astype(jnp.int32) for k in range(N_DEV)]

        def lookup(tbl):
            v = ind[0] * tbl[0]
            for k in range(1, N_DEV):
                v = v + ind[k] * tbl[k]
            return v

        right = lookup([RING[(k + 1) % N_DEV] for k in range(N_DEV)])
        left = lookup([RING[(k - 1) % N_DEV] for k in range(N_DEV)])
        cq = [lookup([RING[(k + r) % N_DEV] for k in range(N_DEV)])
              for r in range(N_DEV)]

        xt_ref[...] = x_ref[...].T

        order = sorted(range(N_DEV), key=QROW)
        for r in order:
            xtp_ref[pl.ds(QROW(r) * chunk, chunk), :] = (
                xt_ref[pl.ds(cq[r] * chunk, chunk), :]
            )

        def piece(lo, hi):
            acc_ref[pl.ds(lo, hi - lo), :] = lax.dot_general(
                xtp_ref[pl.ds(lo, hi - lo), :], dy_ref[...],
                dimension_numbers=(((1,), (0,)), ((), ())),
                preferred_element_type=jnp.float32,
            )

        piece(*_PIECES[0])

        barrier_sem = pltpu.get_barrier_semaphore()
        for nbr in (left, right):
            pl.semaphore_signal(
                barrier_sem, inc=1,
                device_id=(nbr,), device_id_type=pl.DeviceIdType.MESH,
            )
        pl.semaphore_wait(barrier_sem, 2)

        def off(t, s):
            return _STREAMS[t][1] * hrow + s * rh

        def mk_send(t, h, s):
            tgt = right if _STREAMS[t][2] == "R" else left
            row = QROW(_STREAMS[t][3](h)) * chunk + off(t, s)
            return pltpu.make_async_remote_copy(
                src_ref=acc_ref.at[pl.ds(row, rh), :],
                dst_ref=comm_ref.at[t, h, s],
                send_sem=send_sems.at[t, h, s],
                recv_sem=recv_sems.at[t, h, s],
                device_id=(tgt,),
                device_id_type=pl.DeviceIdType.MESH,
            )

        def mk_recv(t, h, s):
            src = left if _STREAMS[t][2] == "R" else right
            return pltpu.make_async_remote_copy(
                src_ref=comm_ref.at[t, h, s],
                dst_ref=comm_ref.at[t, h, s],
                send_sem=send_sems.at[t, h, s],
                recv_sem=recv_sems.at[t, h, s],
                device_id=(src,),
                device_id_type=pl.DeviceIdType.MESH,
            )

        sends = []

        def start(t, h, s):
            rd = mk_send(t, h, s)
            rd.start()
            sends.append(rd)

        for t in range(4):
            for s in range(S):
                start(t, 0, s)

        for lo, hi in _PIECES[1:]:
            piece(lo, hi)

        for h in range(MAX_HOP):
            for t in range(4):
                hops = _STREAMS[t][0]
                if h >= hops:
                    continue
                for s in range(S):
                    mk_recv(t, h, s).wait_recv()
                    if h < hops - 1:
                        row = QROW(_STREAMS[t][4](h)) * chunk + off(t, s)
                        acc_ref[pl.ds(row, rh), :] = (
                            acc_ref[pl.ds(row, rh), :] + comm_ref[t, h, s]
                        )
                        start(t, h + 1, s)

        own = QROW(0) * chunk
        for s in range(S):
            out_ref[pl.ds(off(0, s), rh), :] = (
                acc_ref[pl.ds(own + off(0, s), rh), :]
                + comm_ref[0, 7, s]
                + comm_ref[1, 6, s]
            )
            out_ref[pl.ds(off(2, s), rh), :] = (
                acc_ref[pl.ds(own + off(2, s), rh), :]
                + comm_ref[2, 7, s]
                + comm_ref[3, 6, s]
            )

        for rd in sends:
            rd.wait_send()

    return pl.pallas_call(
        body,
        out_shape=jax.ShapeDtypeStruct((chunk, f), jnp.float32),
        in_specs=[
            pl.BlockSpec(memory_space=pltpu.VMEM),
            pl.BlockSpec(memory_space=pltpu.VMEM),
        ],
        out_specs=pl.BlockSpec(memory_space=pltpu.VMEM),
        scratch_shapes=[
            pltpu.VMEM((d, m), jnp.float32),
            pltpu.VMEM((d, m), jnp.float32),
            pltpu.VMEM((d, f), jnp.float32),
            pltpu.VMEM((4, MAX_HOP, S, rh, f), jnp.float32),
            pltpu.SemaphoreType.DMA((4, MAX_HOP, S)),
            pltpu.SemaphoreType.DMA((4, MAX_HOP, S)),
        ],
        compiler_params=pltpu.CompilerParams(collective_id=0),
    )(x, dy)


# device time: 28487 ns/iter; 1.2192x vs baseline; 1.2192x over previous
import jax
import jax.numpy as jnp
from jax import lax
from jax.experimental import pallas as pl
from jax.experimental.pallas import tpu as pltpu

N_DEV = 16

RING = (0, 1, 5, 9, 13, 14, 10, 6, 2, 3, 7, 11, 15, 12, 8, 4)

S = 2
MAX_HOP = 8

COMM_DTYPE = jnp.bfloat16

def QROW(r):
    return (r - 6) % N_DEV

_PIECES = ((0, 128), (128, 256), (384, 512), (256, 384))

_STREAMS = (
    (8, 0, "R", lambda h: 8 - h, lambda h: 7 - h),
    (7, 0, "L", lambda h: 9 + h, lambda h: 10 + h),
    (8, 1, "L", lambda h: 8 + h, lambda h: 9 + h),
    (7, 1, "R", lambda h: 7 - h, lambda h: 6 - h),
)


def kernel(x, dy):
    m, d = x.shape
    _, f = dy.shape
    chunk = d // N_DEV
    hrow = chunk // 2
    rh = hrow // S

    def body(x_ref, dy_ref, out_ref, xt_ref, xtp_ref, acc_ref, head_ref,
             comm_ref, send_sems, recv_sems):
        my = lax.axis_index("i")

        ind = [(my == RING[k]).astype(jnp.int32) for k in range(N_DEV)]

        def lookup(tbl):
            v = ind[0] * tbl[0]
            for k in range(1, N_DEV):
                v = v + ind[k] * tbl[k]
            return v

        right = lookup([RING[(k + 1) % N_DEV] for k in range(N_DEV)])
        left = lookup([RING[(k - 1) % N_DEV] for k in range(N_DEV)])
        cq = [lookup([RING[(k + r) % N_DEV] for k in range(N_DEV)])
              for r in range(N_DEV)]

        xt_ref[...] = x_ref[...].T

        order = sorted(range(N_DEV), key=QROW)
        for r in order:
            xtp_ref[pl.ds(QROW(r) * chunk, chunk), :] = (
                xt_ref[pl.ds(cq[r] * chunk, chunk), :]
            )

        def piece(lo, hi):
            acc_ref[pl.ds(lo, hi - lo), :] = lax.dot_general(
                xtp_ref[pl.ds(lo, hi - lo), :], dy_ref[...],
                dimension_numbers=(((1,), (0,)), ((), ())),
                preferred_element_type=jnp.float32,
            )

        piece(*_PIECES[0])

        for t in range(4):
            for s in range(S):
                row = QROW(_STREAMS[t][3](0)) * chunk + _STREAMS[t][1] * hrow + s * rh
                head_ref[t, s] = acc_ref[pl.ds(row, rh), :].astype(COMM_DTYPE)

        barrier_sem = pltpu.get_barrier_semaphore()
        for nbr in (left, right):
            pl.semaphore_signal(
                barrier_sem, inc=1,
                device_id=(nbr,), device_id_type=pl.DeviceIdType.MESH,
            )
        pl.semaphore_wait(barrier_sem, 2)

        def off(t, s):
            return _STREAMS[t][1] * hrow + s * rh

        def mk_send(t, h, s):
            tgt = right if _STREAMS[t][2] == "R" else left
            src = head_ref.at[t, s] if h == 0 else comm_ref.at[t, h - 1, s]
            return pltpu.make_async_remote_copy(
                src_ref=src,
                dst_ref=comm_ref.at[t, h, s],
                send_sem=send_sems.at[t, h, s],
                recv_sem=recv_sems.at[t, h, s],
                device_id=(tgt,),
                device_id_type=pl.DeviceIdType.MESH,
            )

        def mk_recv(t, h, s):
            src = left if _STREAMS[t][2] == "R" else right
            return pltpu.make_async_remote_copy(
                src_ref=comm_ref.at[t, h, s],
                dst_ref=comm_ref.at[t, h, s],
                send_sem=send_sems.at[t, h, s],
                recv_sem=recv_sems.at[t, h, s],
                device_id=(src,),
                device_id_type=pl.DeviceIdType.MESH,
            )

        sends = []

        def start(t, h, s):
            rd = mk_send(t, h, s)
            rd.start()
            sends.append(rd)

        for t in range(4):
            for s in range(S):
                start(t, 0, s)

        piece_at = {0: _PIECES[1], 1: _PIECES[2], 2: _PIECES[3]}

        for h in range(MAX_HOP):
            if h in piece_at:
                piece(*piece_at[h])
            for t in range(4):
                hops = _STREAMS[t][0]
                if h >= hops:
                    continue
                for s in range(S):
                    mk_recv(t, h, s).wait_recv()
                    if h < hops - 1:
                        row = QROW(_STREAMS[t][4](h)) * chunk + off(t, s)
                        comm_ref[t, h, s] = (
                            comm_ref[t, h, s].astype(jnp.float32)
                            + acc_ref[pl.ds(row, rh), :]
                        ).astype(COMM_DTYPE)
                        start(t, h + 1, s)

        own = QROW(0) * chunk
        for s in range(S):
            out_ref[pl.ds(off(0, s), rh), :] = (
                acc_ref[pl.ds(own + off(0, s), rh), :]
                + comm_ref[0, 7, s].astype(jnp.float32)
                + comm_ref[1, 6, s].astype(jnp.float32)
            )
            out_ref[pl.ds(off(2, s), rh), :] = (
                acc_ref[pl.ds(own + off(2, s), rh), :]
                + comm_ref[2, 7, s].astype(jnp.float32)
                + comm_ref[3, 6, s].astype(jnp.float32)
            )

        for rd in sends:
            rd.wait_send()

    return pl.pallas_call(
        body,
        out_shape=jax.ShapeDtypeStruct((chunk, f), jnp.float32),
        in_specs=[
            pl.BlockSpec(memory_space=pltpu.VMEM),
            pl.BlockSpec(memory_space=pltpu.VMEM),
        ],
        out_specs=pl.BlockSpec(memory_space=pltpu.VMEM),
        scratch_shapes=[
            pltpu.VMEM((d, m), jnp.float32),
            pltpu.VMEM((d, m), jnp.float32),
            pltpu.VMEM((d, f), jnp.float32),
            pltpu.VMEM((4, S, rh, f), COMM_DTYPE),
            pltpu.VMEM((4, MAX_HOP, S, rh, f), COMM_DTYPE),
            pltpu.SemaphoreType.DMA((4, MAX_HOP, S)),
            pltpu.SemaphoreType.DMA((4, MAX_HOP, S)),
        ],
        compiler_params=pltpu.CompilerParams(collective_id=0),
    )(x, dy)
